# baseline (speedup 1.0000x reference)
import functools

import jax
import jax.numpy as jnp
from jax import lax
from jax.experimental import pallas as pl
from jax.experimental.pallas import tpu as pltpu
from jax.experimental.pallas import tpu_sc as plsc

LEN_FULL_EDGE = 1600000
MAX_TS = 1.0e6
GTAU = 1.0
SPREAD_MASK = 524287

NC = 2
NS = 16
NW = NC * NS
STREAM = 128
CHUNK = 640


def _sc_row_gather_body(feat_hbm, rowid_hbm, out_hbm, rv, buf0, buf1,
                        sem0, sem1, *, n_per_w):
    wid = lax.axis_index("s") * NC + lax.axis_index("c")
    base = wid * n_per_w

    pltpu.sync_copy(rowid_hbm.at[pl.ds(base, n_per_w)], rv)

    n_groups = n_per_w // CHUNK
    spg = CHUNK // STREAM
    bufs = (buf0, buf1)
    sems = (sem0, sem1)

    def fire(g, b):
        for j in range(spg):
            off = g * CHUNK + j * STREAM
            pltpu.async_copy(feat_hbm.at[rv.at[pl.ds(off, STREAM)]],
                             bufs[b].at[pl.ds(j * STREAM, STREAM)], sems[b])

    def drain(b):
        pltpu.make_async_copy(
            feat_hbm.at[pl.ds(0, CHUNK)], bufs[b], sems[b]).wait()

    def write(g, b):
        pltpu.sync_copy(bufs[b], out_hbm.at[pl.ds(base + g * CHUNK, CHUNK)])

    fire(0, 0)

    def pair(g2, _):
        g = g2 * 2
        fire(g + 1, 1)
        drain(0)
        write(g, 0)

        @pl.when(g + 2 < n_groups)
        def _():
            fire(g + 2, 0)

        drain(1)
        write(g + 1, 1)
        return 0

    lax.fori_loop(0, n_groups // 2, pair, 0)


def _run_sc_gather(feat, rowids):
    N = rowids.shape[0]
    H = feat.shape[1]
    n_per_w = N // NW
    mesh = plsc.VectorSubcoreMesh(core_axis_name="c", subcore_axis_name="s")
    body = functools.partial(_sc_row_gather_body, n_per_w=n_per_w)
    f = pl.kernel(
        body,
        mesh=mesh,
        compiler_params=pltpu.CompilerParams(use_tc_tiling_on_sc=False),
        out_type=jax.ShapeDtypeStruct((N, H), jnp.float32),
        scratch_types=[
            pltpu.VMEM((n_per_w,), jnp.int32),
            pltpu.VMEM((CHUNK, H), jnp.float32),
            pltpu.VMEM((CHUNK, H), jnp.float32),
            pltpu.SemaphoreType.DMA,
            pltpu.SemaphoreType.DMA,
        ],
    )
    return f(feat, rowids)


def _lstm_tc(x_ref, wx_ref, wh_ref, b_ref, out_ref):
    T = x_ref.shape[0]
    TB = x_ref.shape[1]
    H = x_ref.shape[2]

    def step(t, carry):
        h, c = carry
        x = x_ref[t]

        def gate(g):
            return (jnp.dot(x, wx_ref[g], preferred_element_type=jnp.float32)
                    + jnp.dot(h, wh_ref[g], preferred_element_type=jnp.float32)
                    + b_ref[g])

        zi = jax.nn.sigmoid(gate(0))
        zf = jax.nn.sigmoid(gate(1))
        zg = jnp.tanh(gate(2))
        zo = jax.nn.sigmoid(gate(3))
        c = zf * c + zi * zg
        h = zo * jnp.tanh(c)
        return (h, c)

    init = (jnp.zeros((TB, H), jnp.float32), jnp.zeros((TB, H), jnp.float32))
    h, _ = jax.lax.fori_loop(0, T, step, init)
    out_ref[...] = h


def _run_lstm(nef, W_ih, W_hh, b_lstm):
    T, B, H = nef.shape
    TB = 256
    wx = jnp.transpose(W_ih.reshape(4, H, H), (0, 2, 1))
    wh = jnp.transpose(W_hh.reshape(4, H, H), (0, 2, 1))
    b4 = b_lstm.reshape(4, H)
    return pl.pallas_call(
        _lstm_tc,
        grid=(B // TB,),
        in_specs=[
            pl.BlockSpec((T, TB, H), lambda i: (0, i, 0)),
            pl.BlockSpec((4, H, H), lambda i: (0, 0, 0)),
            pl.BlockSpec((4, H, H), lambda i: (0, 0, 0)),
            pl.BlockSpec((4, H), lambda i: (0, 0)),
        ],
        out_specs=pl.BlockSpec((TB, H), lambda i: (i, 0)),
        out_shape=jax.ShapeDtypeStruct((B, H), jnp.float32),
    )(nef, wx, wh, b4)


def _logits_gate_tc(cef_ref, vm4_ref, tes_ref, tec_ref, ctx_ref, eps_ref,
                    exp_ref, red_ref, out_ref):
    TBB = cef_ref.shape[0]
    exp_m = exp_ref[...]
    red_m = red_ref[...]
    for i in range(TBB):
        vm = jnp.dot(vm4_ref[i], exp_m, preferred_element_type=jnp.float32)
        prod = (ctx_ref[i] * tec_ref[i]) * (cef_ref[i] * tes_ref[i]) * vm
        logits4 = jnp.dot(prod, red_m, preferred_element_type=jnp.float32)
        bias = 0.0001
        eps = eps_ref[i]
        eps_s = (bias - (1.0 - bias)) * eps + (1.0 - bias)
        gate_in = jnp.log(eps_s) - jnp.log(1.0 - eps_s)
        out_ref[i] = jax.nn.sigmoid((gate_in + logits4) / GTAU)


def _run_logits_gate(cef128, vm4, tes128, tec128, ctx128, eps4):
    B = cef128.shape[0]
    Q = cef128.shape[1]
    H = 32
    TBB = 8
    lane = jnp.arange(128, dtype=jnp.int32)
    exp_m = (lane[None, :] // H == jnp.arange(4, dtype=jnp.int32)[:, None])
    exp_m = exp_m.astype(jnp.float32)
    red_m = jnp.transpose(exp_m)
    return pl.pallas_call(
        _logits_gate_tc,
        grid=(B // TBB,),
        in_specs=[
            pl.BlockSpec((TBB, Q, 128), lambda i: (i, 0, 0)),
            pl.BlockSpec((TBB, Q, 4), lambda i: (i, 0, 0)),
            pl.BlockSpec((TBB, Q, 128), lambda i: (i, 0, 0)),
            pl.BlockSpec((TBB, Q, 128), lambda i: (i, 0, 0)),
            pl.BlockSpec((TBB, 128), lambda i: (i, 0)),
            pl.BlockSpec((TBB, Q, 4), lambda i: (i, 0, 0)),
            pl.BlockSpec((4, 128), lambda i: (0, 0)),
            pl.BlockSpec((128, 4), lambda i: (0, 0)),
        ],
        out_specs=pl.BlockSpec((TBB, Q, 4), lambda i: (i, 0, 0)),
        out_shape=jax.ShapeDtypeStruct((B, Q, 4), jnp.float32),
    )(cef128, vm4, tes128, tec128, ctx128, eps4, exp_m, red_m)


def kernel(train_edge_feat, candidate_ts, ts_aug, eps, W_ih, W_hh, b_lstm,
           w_t, b_t, train_e_idx_l, neighbor_edge_idx, candidate_edge_idx):
    E = train_edge_feat.shape[0]
    H = train_edge_feat.shape[1]
    B, RNN_NN = neighbor_edge_idx.shape
    CAN = candidate_edge_idx.shape[1]

    pos = jnp.full((LEN_FULL_EDGE + 1,), -1, dtype=jnp.int32)
    pos = pos.at[train_e_idx_l].set(jnp.arange(E, dtype=jnp.int32))

    npos = jnp.take(pos, neighbor_edge_idx.reshape(-1), axis=0)
    nspread = jnp.arange(npos.shape[0], dtype=jnp.int32) & SPREAD_MASK
    nrow = jnp.where(npos >= 0, npos, nspread)
    nef_raw = _run_sc_gather(train_edge_feat, nrow)
    nmask = (npos >= 0).astype(jnp.float32)
    nef = (nef_raw * nmask[:, None]).reshape(B, RNN_NN, H).transpose(1, 0, 2)

    context_vec = _run_lstm(nef, W_ih, W_hh, b_lstm)

    cpos = jnp.take(pos, candidate_edge_idx.reshape(-1), axis=0)
    cspread = jnp.arange(cpos.shape[0], dtype=jnp.int32) & SPREAD_MASK
    crow = jnp.where(cpos >= 0, cpos, cspread)
    cef_raw = _run_sc_gather(train_edge_feat, crow)
    vm4 = (cpos >= 0).astype(jnp.float32).reshape(B, CAN // 4, 4)

    c_ts = candidate_ts * MAX_TS
    a_ts = ts_aug * MAX_TS
    delta_ts_sample = a_ts - c_ts
    delta_ts_sample_context = a_ts - MAX_TS
    te_sample = jnp.cos(delta_ts_sample[..., None] * w_t + b_t)
    te_context = jnp.cos(delta_ts_sample_context[..., None] * w_t + b_t)

    eps4 = eps.reshape(B, CAN // 4, 4)
    cef128 = cef_raw.reshape(B, CAN // 4, 4 * H)
    tes128 = te_sample.reshape(B, CAN // 4, 4 * H)
    tec128 = te_context.reshape(B, CAN // 4, 4 * H)
    ctx128 = jnp.tile(context_vec, (1, 4))

    gate4 = _run_logits_gate(cef128, vm4, tes128, tec128, ctx128, eps4)
    return gate4.reshape(B, CAN)


# speedup vs baseline: 1.1317x; 1.1317x over previous
import functools

import jax
import jax.numpy as jnp
from jax import lax
from jax.experimental import pallas as pl
from jax.experimental.pallas import tpu as pltpu
from jax.experimental.pallas import tpu_sc as plsc

LEN_FULL_EDGE = 1600000
MAX_TS = 1.0e6
GTAU = 1.0
SPREAD_MASK = 524287

NC = 2
NS = 16
NW = NC * NS
STREAM = 128
CHUNK = 640


def _sc_row_gather_body(feat_hbm, rowid_hbm, out_hbm, rv, buf0, buf1,
                        sem0, sem1, *, n_per_w):
    wid = lax.axis_index("s") * NC + lax.axis_index("c")
    base = wid * n_per_w

    pltpu.sync_copy(rowid_hbm.at[pl.ds(base, n_per_w)], rv)

    n_groups = n_per_w // CHUNK
    spg = CHUNK // STREAM
    bufs = (buf0, buf1)
    sems = (sem0, sem1)

    def fire(g, b):
        for j in range(spg):
            off = g * CHUNK + j * STREAM
            pltpu.async_copy(feat_hbm.at[rv.at[pl.ds(off, STREAM)]],
                             bufs[b].at[pl.ds(j * STREAM, STREAM)], sems[b])

    def drain(b):
        pltpu.make_async_copy(
            feat_hbm.at[pl.ds(0, CHUNK)], bufs[b], sems[b]).wait()

    def write(g, b):
        pltpu.sync_copy(bufs[b], out_hbm.at[pl.ds(base + g * CHUNK, CHUNK)])

    fire(0, 0)

    def pair(g2, _):
        g = g2 * 2
        fire(g + 1, 1)
        drain(0)
        write(g, 0)

        @pl.when(g + 2 < n_groups)
        def _():
            fire(g + 2, 0)

        drain(1)
        write(g + 1, 1)
        return 0

    lax.fori_loop(0, n_groups // 2, pair, 0)


def _run_sc_gather(feat, rowids):
    N = rowids.shape[0]
    H = feat.shape[1]
    n_per_w = N // NW
    mesh = plsc.VectorSubcoreMesh(core_axis_name="c", subcore_axis_name="s")
    body = functools.partial(_sc_row_gather_body, n_per_w=n_per_w)
    f = pl.kernel(
        body,
        mesh=mesh,
        compiler_params=pltpu.CompilerParams(use_tc_tiling_on_sc=False),
        out_type=jax.ShapeDtypeStruct((N, H), jnp.float32),
        scratch_types=[
            pltpu.VMEM((n_per_w,), jnp.int32),
            pltpu.VMEM((CHUNK, H), jnp.float32),
            pltpu.VMEM((CHUNK, H), jnp.float32),
            pltpu.SemaphoreType.DMA,
            pltpu.SemaphoreType.DMA,
        ],
    )
    return f(feat, rowids)


def _lstm_tc(x_ref, wx_ref, wh_ref, b_ref, out_ref):
    T = x_ref.shape[0]
    TB = x_ref.shape[1]
    H = x_ref.shape[2]

    def step(t, carry):
        h, c = carry
        x = x_ref[t]

        def gate(g):
            return (jnp.dot(x, wx_ref[g], preferred_element_type=jnp.float32)
                    + jnp.dot(h, wh_ref[g], preferred_element_type=jnp.float32)
                    + b_ref[g])

        zi = jax.nn.sigmoid(gate(0))
        zf = jax.nn.sigmoid(gate(1))
        zg = jnp.tanh(gate(2))
        zo = jax.nn.sigmoid(gate(3))
        c = zf * c + zi * zg
        h = zo * jnp.tanh(c)
        return (h, c)

    init = (jnp.zeros((TB, H), jnp.float32), jnp.zeros((TB, H), jnp.float32))
    h, _ = jax.lax.fori_loop(0, T, step, init)
    out_ref[...] = h


def _run_lstm(nef, W_ih, W_hh, b_lstm):
    T, B, H = nef.shape
    TB = 256
    wx = jnp.transpose(W_ih.reshape(4, H, H), (0, 2, 1))
    wh = jnp.transpose(W_hh.reshape(4, H, H), (0, 2, 1))
    b4 = b_lstm.reshape(4, H)
    return pl.pallas_call(
        _lstm_tc,
        grid=(B // TB,),
        in_specs=[
            pl.BlockSpec((T, TB, H), lambda i: (0, i, 0)),
            pl.BlockSpec((4, H, H), lambda i: (0, 0, 0)),
            pl.BlockSpec((4, H, H), lambda i: (0, 0, 0)),
            pl.BlockSpec((4, H), lambda i: (0, 0)),
        ],
        out_specs=pl.BlockSpec((TB, H), lambda i: (i, 0)),
        out_shape=jax.ShapeDtypeStruct((B, H), jnp.float32),
    )(nef, wx, wh, b4)


def _logits_gate_tc(cef_ref, vm4_ref, t_ref, eps_ref,
                    exp_ref, red_ref, out_ref):
    TBB = cef_ref.shape[0]
    exp_m = exp_ref[...]
    red_m = red_ref[...]
    for i in range(TBB):
        vm = jnp.dot(vm4_ref[i], exp_m, preferred_element_type=jnp.float32)
        prod = t_ref[i] * cef_ref[i] * vm
        logits4 = jnp.dot(prod, red_m, preferred_element_type=jnp.float32)
        bias = 0.0001
        eps = eps_ref[i]
        eps_s = (bias - (1.0 - bias)) * eps + (1.0 - bias)
        gate_in = jnp.log(eps_s) - jnp.log(1.0 - eps_s)
        out_ref[i] = jax.nn.sigmoid((gate_in + logits4) / GTAU)


def _run_logits_gate(cef128, vm4, t128, eps4):
    B = cef128.shape[0]
    Q = cef128.shape[1]
    H = 32
    TBB = 8
    lane = jnp.arange(128, dtype=jnp.int32)
    exp_m = (lane[None, :] // H == jnp.arange(4, dtype=jnp.int32)[:, None])
    exp_m = exp_m.astype(jnp.float32)
    red_m = jnp.transpose(exp_m)
    return pl.pallas_call(
        _logits_gate_tc,
        grid=(B // TBB,),
        in_specs=[
            pl.BlockSpec((TBB, Q, 128), lambda i: (i, 0, 0)),
            pl.BlockSpec((TBB, Q, 4), lambda i: (i, 0, 0)),
            pl.BlockSpec((TBB, Q, 128), lambda i: (i, 0, 0)),
            pl.BlockSpec((TBB, Q, 4), lambda i: (i, 0, 0)),
            pl.BlockSpec((4, 128), lambda i: (0, 0)),
            pl.BlockSpec((128, 4), lambda i: (0, 0)),
        ],
        out_specs=pl.BlockSpec((TBB, Q, 4), lambda i: (i, 0, 0)),
        out_shape=jax.ShapeDtypeStruct((B, Q, 4), jnp.float32),
    )(cef128, vm4, t128, eps4, exp_m, red_m)


def kernel(train_edge_feat, candidate_ts, ts_aug, eps, W_ih, W_hh, b_lstm,
           w_t, b_t, train_e_idx_l, neighbor_edge_idx, candidate_edge_idx):
    E = train_edge_feat.shape[0]
    H = train_edge_feat.shape[1]
    B, RNN_NN = neighbor_edge_idx.shape
    CAN = candidate_edge_idx.shape[1]

    pos = jnp.full((LEN_FULL_EDGE + 1,), -1, dtype=jnp.int32)
    pos = pos.at[train_e_idx_l].max(jnp.arange(E, dtype=jnp.int32))

    npos = jnp.take(pos, neighbor_edge_idx.reshape(-1), axis=0)
    nspread = jnp.arange(npos.shape[0], dtype=jnp.int32) & SPREAD_MASK
    nrow = jnp.where(npos >= 0, npos, nspread)
    nef_raw = _run_sc_gather(train_edge_feat, nrow)
    nmask = (npos >= 0).astype(jnp.float32)
    nef = (nef_raw * nmask[:, None]).reshape(B, RNN_NN, H).transpose(1, 0, 2)

    context_vec = _run_lstm(nef, W_ih, W_hh, b_lstm)

    cpos = jnp.take(pos, candidate_edge_idx.reshape(-1), axis=0)
    cspread = jnp.arange(cpos.shape[0], dtype=jnp.int32) & SPREAD_MASK
    crow = jnp.where(cpos >= 0, cpos, cspread)
    cef_raw = _run_sc_gather(train_edge_feat, crow)
    vm4 = (cpos >= 0).astype(jnp.float32).reshape(B, CAN // 4, 4)

    c_ts = candidate_ts * MAX_TS
    a_ts = ts_aug * MAX_TS
    delta_ts_sample = a_ts - c_ts
    delta_ts_sample_context = a_ts - MAX_TS
    te_sample = jnp.cos(delta_ts_sample[..., None] * w_t + b_t)
    te_context = jnp.cos(delta_ts_sample_context[..., None] * w_t + b_t)
    t_all = (context_vec[:, None, :] * te_context) * te_sample

    eps4 = eps.reshape(B, CAN // 4, 4)
    cef128 = cef_raw.reshape(B, CAN // 4, 4 * H)
    t128 = t_all.reshape(B, CAN // 4, 4 * H)

    gate4 = _run_logits_gate(cef128, vm4, t128, eps4)
    return gate4.reshape(B, CAN)


# speedup vs baseline: 1.1844x; 1.0465x over previous
import functools

import jax
import jax.numpy as jnp
from jax import lax
from jax.experimental import pallas as pl
from jax.experimental.pallas import tpu as pltpu
from jax.experimental.pallas import tpu_sc as plsc

LEN_FULL_EDGE = 1600000
MAX_TS = 1.0e6
GTAU = 1.0
SPREAD_MASK = 524287

NC = 2
NS = 16
NW = NC * NS
STREAM = 128
CHUNK = 640


def _sc_row_gather_body(feat_hbm, rowid_hbm, out_hbm, rv, buf0, buf1,
                        sem0, sem1, *, n_per_w):
    wid = lax.axis_index("s") * NC + lax.axis_index("c")
    base = wid * n_per_w

    pltpu.sync_copy(rowid_hbm.at[pl.ds(base, n_per_w)], rv)

    n_groups = n_per_w // CHUNK
    spg = CHUNK // STREAM
    bufs = (buf0, buf1)
    sems = (sem0, sem1)

    def fire(g, b):
        for j in range(spg):
            off = g * CHUNK + j * STREAM
            pltpu.async_copy(feat_hbm.at[rv.at[pl.ds(off, STREAM)]],
                             bufs[b].at[pl.ds(j * STREAM, STREAM)], sems[b])

    def drain(b):
        pltpu.make_async_copy(
            feat_hbm.at[pl.ds(0, CHUNK)], bufs[b], sems[b]).wait()

    def write(g, b):
        pltpu.sync_copy(bufs[b], out_hbm.at[pl.ds(base + g * CHUNK, CHUNK)])

    fire(0, 0)

    def pair(g2, _):
        g = g2 * 2
        fire(g + 1, 1)
        drain(0)
        write(g, 0)

        @pl.when(g + 2 < n_groups)
        def _():
            fire(g + 2, 0)

        drain(1)
        write(g + 1, 1)
        return 0

    lax.fori_loop(0, n_groups // 2, pair, 0)


def _run_sc_gather(feat, rowids):
    N = rowids.shape[0]
    H = feat.shape[1]
    n_per_w = N // NW
    mesh = plsc.VectorSubcoreMesh(core_axis_name="c", subcore_axis_name="s")
    body = functools.partial(_sc_row_gather_body, n_per_w=n_per_w)
    f = pl.kernel(
        body,
        mesh=mesh,
        compiler_params=pltpu.CompilerParams(use_tc_tiling_on_sc=False),
        out_type=jax.ShapeDtypeStruct((N, H), jnp.float32),
        scratch_types=[
            pltpu.VMEM((n_per_w,), jnp.int32),
            pltpu.VMEM((CHUNK, H), jnp.float32),
            pltpu.VMEM((CHUNK, H), jnp.float32),
            pltpu.SemaphoreType.DMA,
            pltpu.SemaphoreType.DMA,
        ],
    )
    return f(feat, rowids)


def _lstm_tc(x_ref, wx_ref, wh_ref, b_ref, out_ref):
    T = x_ref.shape[0]
    TB = x_ref.shape[1]
    H = x_ref.shape[2]

    def step(t, carry):
        h, c = carry
        x = x_ref[t]

        def gate(g):
            return (jnp.dot(x, wx_ref[g], preferred_element_type=jnp.float32)
                    + jnp.dot(h, wh_ref[g], preferred_element_type=jnp.float32)
                    + b_ref[g])

        zi = jax.nn.sigmoid(gate(0))
        zf = jax.nn.sigmoid(gate(1))
        zg = jnp.tanh(gate(2))
        zo = jax.nn.sigmoid(gate(3))
        c = zf * c + zi * zg
        h = zo * jnp.tanh(c)
        return (h, c)

    init = (jnp.zeros((TB, H), jnp.float32), jnp.zeros((TB, H), jnp.float32))
    h, _ = jax.lax.fori_loop(0, T, step, init)
    out_ref[...] = h


def _run_lstm(nef, W_ih, W_hh, b_lstm):
    T, B, H = nef.shape
    TB = 256
    wx = jnp.transpose(W_ih.reshape(4, H, H), (0, 2, 1))
    wh = jnp.transpose(W_hh.reshape(4, H, H), (0, 2, 1))
    b4 = b_lstm.reshape(4, H)
    return pl.pallas_call(
        _lstm_tc,
        grid=(B // TB,),
        in_specs=[
            pl.BlockSpec((T, TB, H), lambda i: (0, i, 0)),
            pl.BlockSpec((4, H, H), lambda i: (0, 0, 0)),
            pl.BlockSpec((4, H, H), lambda i: (0, 0, 0)),
            pl.BlockSpec((4, H), lambda i: (0, 0)),
        ],
        out_specs=pl.BlockSpec((TB, H), lambda i: (i, 0)),
        out_shape=jax.ShapeDtypeStruct((B, H), jnp.float32),
    )(nef, wx, wh, b4)


def _logits_gate_tc(cef_ref, vm4_ref, t_ref, ctx_ref, eps_ref,
                    exp_ref, red_ref, out_ref):
    TBB = cef_ref.shape[0]
    exp_m = exp_ref[...]
    red_m = red_ref[...]
    for i in range(TBB):
        vm = jnp.dot(vm4_ref[i], exp_m, preferred_element_type=jnp.float32)
        prod = (ctx_ref[i] * t_ref[i]) * cef_ref[i] * vm
        logits4 = jnp.dot(prod, red_m, preferred_element_type=jnp.float32)
        bias = 0.0001
        eps = eps_ref[i]
        eps_s = (bias - (1.0 - bias)) * eps + (1.0 - bias)
        gate_in = jnp.log(eps_s) - jnp.log(1.0 - eps_s)
        out_ref[i] = jax.nn.sigmoid((gate_in + logits4) / GTAU)


def _run_logits_gate(cef128, vm4, t128, ctx128, eps4):
    B = cef128.shape[0]
    Q = cef128.shape[1]
    H = 32
    TBB = 8
    lane = jnp.arange(128, dtype=jnp.int32)
    exp_m = (lane[None, :] // H == jnp.arange(4, dtype=jnp.int32)[:, None])
    exp_m = exp_m.astype(jnp.float32)
    red_m = jnp.transpose(exp_m)
    return pl.pallas_call(
        _logits_gate_tc,
        grid=(B // TBB,),
        in_specs=[
            pl.BlockSpec((TBB, Q, 128), lambda i: (i, 0, 0)),
            pl.BlockSpec((TBB, Q, 4), lambda i: (i, 0, 0)),
            pl.BlockSpec((TBB, Q, 128), lambda i: (i, 0, 0)),
            pl.BlockSpec((TBB, 128), lambda i: (i, 0)),
            pl.BlockSpec((TBB, Q, 4), lambda i: (i, 0, 0)),
            pl.BlockSpec((4, 128), lambda i: (0, 0)),
            pl.BlockSpec((128, 4), lambda i: (0, 0)),
        ],
        out_specs=pl.BlockSpec((TBB, Q, 4), lambda i: (i, 0, 0)),
        out_shape=jax.ShapeDtypeStruct((B, Q, 4), jnp.float32),
    )(cef128, vm4, t128, ctx128, eps4, exp_m, red_m)


def kernel(train_edge_feat, candidate_ts, ts_aug, eps, W_ih, W_hh, b_lstm,
           w_t, b_t, train_e_idx_l, neighbor_edge_idx, candidate_edge_idx):
    E = train_edge_feat.shape[0]
    H = train_edge_feat.shape[1]
    B, RNN_NN = neighbor_edge_idx.shape
    CAN = candidate_edge_idx.shape[1]

    pos = jnp.full((LEN_FULL_EDGE + 1,), -1, dtype=jnp.int32)
    pos = pos.at[train_e_idx_l].max(jnp.arange(E, dtype=jnp.int32))

    npos = jnp.take(pos, neighbor_edge_idx.reshape(-1), axis=0)
    nspread = jnp.arange(npos.shape[0], dtype=jnp.int32) & SPREAD_MASK
    nrow = jnp.where(npos >= 0, npos, nspread)
    nef_raw = _run_sc_gather(train_edge_feat, nrow)
    nmask = (npos >= 0).astype(jnp.float32)
    nef = (nef_raw * nmask[:, None]).reshape(B, RNN_NN, H).transpose(1, 0, 2)

    context_vec = _run_lstm(nef, W_ih, W_hh, b_lstm)

    cpos = jnp.take(pos, candidate_edge_idx.reshape(-1), axis=0)
    cspread = jnp.arange(cpos.shape[0], dtype=jnp.int32) & SPREAD_MASK
    crow = jnp.where(cpos >= 0, cpos, cspread)
    cef_raw = _run_sc_gather(train_edge_feat, crow)
    vm4 = (cpos >= 0).astype(jnp.float32).reshape(B, CAN // 4, 4)

    c_ts = candidate_ts * MAX_TS
    a_ts = ts_aug * MAX_TS
    delta_ts_sample = a_ts - c_ts
    delta_ts_sample_context = a_ts - MAX_TS
    te_sample = jnp.cos(delta_ts_sample[..., None] * w_t + b_t)
    te_context = jnp.cos(delta_ts_sample_context[..., None] * w_t + b_t)
    t_all = te_context * te_sample

    eps4 = eps.reshape(B, CAN // 4, 4)
    cef128 = cef_raw.reshape(B, CAN // 4, 4 * H)
    t128 = t_all.reshape(B, CAN // 4, 4 * H)
    ctx128 = jnp.tile(context_vec, (1, 4))

    gate4 = _run_logits_gate(cef128, vm4, t128, ctx128, eps4)
    return gate4.reshape(B, CAN)
